# baseline (speedup 1.0000x reference)
import functools

import jax
import jax.numpy as jnp
from jax import lax
from jax.experimental import pallas as pl
from jax.experimental.pallas import tpu as pltpu
from jax.experimental.pallas import tpu_sc as plsc

_VOCAB = 1000
_EMB = 128
_BATCH = 16384

_NC = 2
_NS = 16
_NW = _NC * _NS
_BPW = _BATCH // _NW
_CHUNK = 128
_NCHUNK = _BPW // _CHUNK

_BM = 512


def _sc_gather_body(table_hbm, x_hbm, emb_hbm, idx_v, rows_v, sem):
    wid = lax.axis_index("s") * _NC + lax.axis_index("c")
    base = wid * _BPW
    pltpu.sync_copy(x_hbm.at[pl.ds(base, _BPW)], idx_v)
    for c in range(_NCHUNK):
        ids = idx_v.at[pl.ds(c * _CHUNK, _CHUNK)]
        pltpu.async_copy(table_hbm.at[ids], rows_v, sem).wait()
        pltpu.sync_copy(rows_v, emb_hbm.at[pl.ds(base + c * _CHUNK, _CHUNK)])


@functools.cache
def _sc_gather():
    return pl.kernel(
        _sc_gather_body,
        out_type=jax.ShapeDtypeStruct((_BATCH, _EMB), jnp.float32),
        mesh=plsc.VectorSubcoreMesh(core_axis_name="c", subcore_axis_name="s",
                                    num_cores=_NC, num_subcores=_NS),
        scratch_types=[
            pltpu.VMEM((_BPW,), jnp.int32),
            pltpu.VMEM((_CHUNK, _EMB), jnp.float32),
            pltpu.SemaphoreType.DMA,
        ],
    )


def _tc_matmul_kernel(e_ref, wt_ref, b_ref, o_ref):
    o_ref[...] = (
        jnp.dot(e_ref[...], wt_ref[...], preferred_element_type=jnp.float32)
        + b_ref[0:1, :]
    )


def _tc_matmul(emb, Wt, b2):
    return pl.pallas_call(
        _tc_matmul_kernel,
        grid=(_BATCH // _BM,),
        in_specs=[
            pl.BlockSpec((_BM, _EMB), lambda i: (i, 0)),
            pl.BlockSpec((_EMB, _VOCAB), lambda i: (0, 0)),
            pl.BlockSpec((1, _VOCAB), lambda i: (0, 0)),
        ],
        out_specs=pl.BlockSpec((_BM, _VOCAB), lambda i: (i, 0)),
        out_shape=jax.ShapeDtypeStruct((_BATCH, _VOCAB), jnp.float32),
    )(emb, Wt, b2)


@jax.jit
def kernel(x, table, W, b):
    xi = x.astype(jnp.int32)
    emb = _sc_gather()(table, xi)
    out = _tc_matmul(emb, W.T, b.reshape(1, _VOCAB))
    return out, emb


# speedup vs baseline: 1.1001x; 1.1001x over previous
import functools

import jax
import jax.numpy as jnp
from jax import lax
from jax.experimental import pallas as pl
from jax.experimental.pallas import tpu as pltpu
from jax.experimental.pallas import tpu_sc as plsc

_VOCAB = 1000
_EMB = 128
_BATCH = 16384

_NC = 2
_NS = 16
_NW = _NC * _NS
_BPW = _BATCH // _NW
_CHUNK = 128
_NCHUNK = _BPW // _CHUNK

_BM = 2048


def _sc_gather_body(table_hbm, x_hbm, emb_hbm, idx_v, rows_v, sem):
    wid = lax.axis_index("s") * _NC + lax.axis_index("c")
    base = wid * _BPW
    pltpu.sync_copy(x_hbm.at[pl.ds(base, _BPW)], idx_v)
    for c in range(_NCHUNK):
        ids = idx_v.at[pl.ds(c * _CHUNK, _CHUNK)]
        pltpu.async_copy(table_hbm.at[ids], rows_v, sem).wait()
        pltpu.sync_copy(rows_v, emb_hbm.at[pl.ds(base + c * _CHUNK, _CHUNK)])


@functools.cache
def _sc_gather():
    return pl.kernel(
        _sc_gather_body,
        out_type=jax.ShapeDtypeStruct((_BATCH, _EMB), jnp.float32),
        mesh=plsc.VectorSubcoreMesh(core_axis_name="c", subcore_axis_name="s",
                                    num_cores=_NC, num_subcores=_NS),
        scratch_types=[
            pltpu.VMEM((_BPW,), jnp.int32),
            pltpu.VMEM((_CHUNK, _EMB), jnp.float32),
            pltpu.SemaphoreType.DMA,
        ],
    )


def _tc_matmul_kernel(e_ref, wt_ref, b_ref, o_ref):
    o_ref[...] = (
        jnp.dot(e_ref[...], wt_ref[...], preferred_element_type=jnp.float32)
        + b_ref[0:1, :]
    )


def _tc_matmul(emb, Wt, b2):
    return pl.pallas_call(
        _tc_matmul_kernel,
        grid=(_BATCH // _BM,),
        in_specs=[
            pl.BlockSpec((_BM, _EMB), lambda i: (i, 0)),
            pl.BlockSpec((_EMB, _VOCAB), lambda i: (0, 0)),
            pl.BlockSpec((1, _VOCAB), lambda i: (0, 0)),
        ],
        out_specs=pl.BlockSpec((_BM, _VOCAB), lambda i: (i, 0)),
        out_shape=jax.ShapeDtypeStruct((_BATCH, _VOCAB), jnp.float32),
    )(emb, Wt, b2)


@jax.jit
def kernel(x, table, W, b):
    xi = x.astype(jnp.int32)
    emb = _sc_gather()(table, xi)
    out = _tc_matmul(emb, W.T, b.reshape(1, _VOCAB))
    return out, emb
